# baseline (speedup 1.0000x reference)
import functools

import jax
import jax.numpy as jnp
from jax import lax
from jax.experimental import pallas as pl
from jax.experimental.pallas import tpu as pltpu
from jax.experimental.pallas import tpu_sc as plsc

NUM_EMBEDDING = 1000000
EMBEDDING_DIM = 64
IBLK = 128
TILE = 8 * IBLK

_INFO = plsc.get_sparse_core_info()
_NC = _INFO.num_cores
_NS = _INFO.num_subcores
_NW = _NC * _NS


def _make_lookup(n_tokens, n_per):
    i_per_w = n_tokens // _NW
    nblk = i_per_w // IBLK
    n_iblk = n_tokens // IBLK
    n_jobs = n_per * nblk
    dtiles = EMBEDDING_DIM // 8
    mesh = plsc.VectorSubcoreMesh(core_axis_name="c", subcore_axis_name="s")

    @functools.partial(
        pl.kernel,
        mesh=mesh,
        out_type=jax.ShapeDtypeStruct((n_tokens * n_per * EMBEDDING_DIM,),
                                      jnp.float32),
        scratch_types=[
            pltpu.VMEM((n_per, i_per_w), jnp.int32),
            pltpu.VMEM((2, IBLK, EMBEDDING_DIM), jnp.float32),
            pltpu.VMEM((2, EMBEDDING_DIM * IBLK), jnp.float32),
            pltpu.SemaphoreType.DMA,
            pltpu.SemaphoreType.DMA,
        ],
        compiler_params=pltpu.CompilerParams(use_tc_tiling_on_sc=False,
                                             needs_layout_passes=False,
                                             disable_bounds_checks=True),
    )
    def lookup_kernel(tok_hbm, table_hbm, out_hbm, idx_v, rows_v, t_v,
                      gsem, wsem):
        wid = lax.axis_index("s") * _NC + lax.axis_index("c")
        i0w = wid * i_per_w
        pltpu.sync_copy(tok_hbm.at[:, pl.ds(i0w, i_per_w)], idx_v)
        lanes = lax.iota(jnp.int32, 16)

        def fire_gather(job, buf):
            j = lax.div(job, nblk)
            ib = lax.rem(job, nblk)
            pltpu.async_copy(
                table_hbm.at[idx_v.at[j, pl.ds(ib * IBLK, IBLK)]],
                rows_v.at[buf],
                gsem,
            )

        def drain_gather(buf):
            pltpu.make_async_copy(
                table_hbm.at[pl.ds(0, IBLK)],
                rows_v.at[buf],
                gsem,
            ).wait()

        def drain_writebacks():
            pltpu.make_async_copy(
                t_v.at[0],
                out_hbm.at[pl.ds(0, EMBEDDING_DIM * IBLK)],
                wsem,
            ).wait()

        def transpose_into(buf):
            @plsc.parallel_loop(0, EMBEDDING_DIM, 1, unroll=8)
            def _(d):
                dvec = jnp.full((16,), 0, jnp.int32) + d
                base = d * IBLK
                for ic in range(IBLK // 16):
                    vals = plsc.load_gather(
                        rows_v.at[buf],
                        [ic * 16 + lanes, dvec],
                    )
                    t_v[buf, pl.ds(base + ic * 16, 16)] = vals

        def fire_writebacks(job, buf):
            j = lax.div(job, nblk)
            ib = lax.rem(job, nblk)
            gblk = wid * nblk + ib
            for db in range(dtiles):
                pltpu.async_copy(
                    t_v.at[buf, pl.ds(db * TILE, TILE)],
                    out_hbm.at[pl.ds(((j * dtiles + db) * n_iblk + gblk)
                                     * TILE, TILE)],
                    wsem,
                )

        fire_gather(0, 0)
        fire_writebacks(0, 1)

        def job_body(job, carry):
            cur = lax.rem(job, 2)
            nxt = 1 - cur
            drain_gather(cur)
            nxt_job = lax.min(job + 1, n_jobs - 1)
            fire_gather(nxt_job, nxt)
            transpose_into(cur)
            drain_writebacks()
            fire_writebacks(job, cur)
            return carry

        lax.fori_loop(0, n_jobs, job_body, 0)
        drain_gather(lax.rem(n_jobs, 2))
        drain_writebacks()

    return lookup_kernel


def kernel(token_ids, weight):
    n_tokens, n_per = token_ids.shape
    tok2 = token_ids.T.astype(jnp.int32)
    out1d = _make_lookup(n_tokens, n_per)(tok2, weight)
    o5 = out1d.reshape(n_per, EMBEDDING_DIM // 8, n_tokens // IBLK, 8, IBLK)
    return o5.transpose(2, 4, 0, 1, 3).reshape(n_tokens, n_per, EMBEDDING_DIM)


# speedup vs baseline: 1.0449x; 1.0449x over previous
import functools

import jax
import jax.numpy as jnp
from jax import lax
from jax.experimental import pallas as pl
from jax.experimental.pallas import tpu as pltpu
from jax.experimental.pallas import tpu_sc as plsc

NUM_EMBEDDING = 1000000
EMBEDDING_DIM = 64
IBLK = 128
TILE = 8 * IBLK

_INFO = plsc.get_sparse_core_info()
_NC = _INFO.num_cores
_NS = _INFO.num_subcores
_NW = _NC * _NS


def _make_lookup(n_tokens, n_per):
    i_per_w = n_tokens // _NW
    nblk = i_per_w // IBLK
    n_iblk = n_tokens // IBLK
    n_jobs = n_per * nblk
    dtiles = EMBEDDING_DIM // 8
    mesh = plsc.VectorSubcoreMesh(core_axis_name="c", subcore_axis_name="s")

    @functools.partial(
        pl.kernel,
        mesh=mesh,
        out_type=jax.ShapeDtypeStruct((n_tokens * n_per * EMBEDDING_DIM,),
                                      jnp.float32),
        scratch_types=[
            pltpu.VMEM((n_per, i_per_w), jnp.int32),
            pltpu.VMEM((2, IBLK, EMBEDDING_DIM), jnp.float32),
            pltpu.VMEM((2, EMBEDDING_DIM * IBLK), jnp.float32),
            pltpu.SemaphoreType.DMA,
            pltpu.SemaphoreType.DMA,
        ],
        compiler_params=pltpu.CompilerParams(use_tc_tiling_on_sc=False,
                                             needs_layout_passes=False,
                                             disable_bounds_checks=True),
    )
    def lookup_kernel(tok_hbm, table_hbm, out_hbm, idx_v, rows_v, t_v,
                      gsem, wsem):
        wid = lax.axis_index("s") * _NC + lax.axis_index("c")
        i0w = wid * i_per_w
        pltpu.sync_copy(tok_hbm.at[:, pl.ds(i0w, i_per_w)], idx_v)
        lanes = lax.iota(jnp.int32, 16)

        def fire_gather(job, buf):
            j = lax.div(job, nblk)
            ib = lax.rem(job, nblk)
            pltpu.async_copy(
                table_hbm.at[idx_v.at[j, pl.ds(ib * IBLK, IBLK)]],
                rows_v.at[buf],
                gsem,
            )

        def drain_gather(buf):
            pltpu.make_async_copy(
                table_hbm.at[pl.ds(0, IBLK)],
                rows_v.at[buf],
                gsem,
            ).wait()

        def drain_writebacks():
            pltpu.make_async_copy(
                t_v.at[0],
                out_hbm.at[pl.ds(0, EMBEDDING_DIM * IBLK)],
                wsem,
            ).wait()

        def transpose_into(buf):
            @plsc.parallel_loop(0, EMBEDDING_DIM, 1, unroll=4)
            def _(d):
                dvec = jnp.full((16,), 0, jnp.int32) + d
                base = d * IBLK
                for ic in range(IBLK // 16):
                    vals = plsc.load_gather(
                        rows_v.at[buf],
                        [ic * 16 + lanes, dvec],
                    )
                    t_v[buf, pl.ds(base + ic * 16, 16)] = vals

        def fire_writebacks(job, buf):
            j = lax.div(job, nblk)
            ib = lax.rem(job, nblk)
            gblk = wid * nblk + ib
            for db in range(dtiles):
                pltpu.async_copy(
                    t_v.at[buf, pl.ds(db * TILE, TILE)],
                    out_hbm.at[pl.ds(((j * dtiles + db) * n_iblk + gblk)
                                     * TILE, TILE)],
                    wsem,
                )

        fire_gather(0, 0)
        fire_writebacks(0, 1)

        def job_body(job, carry):
            cur = lax.rem(job, 2)
            nxt = 1 - cur
            drain_gather(cur)
            nxt_job = lax.min(job + 1, n_jobs - 1)
            fire_gather(nxt_job, nxt)
            transpose_into(cur)
            drain_writebacks()
            fire_writebacks(job, cur)
            return carry

        lax.fori_loop(0, n_jobs, job_body, 0)
        drain_gather(lax.rem(n_jobs, 2))
        drain_writebacks()

    return lookup_kernel


def kernel(token_ids, weight):
    n_tokens, n_per = token_ids.shape
    tok2 = token_ids.T.astype(jnp.int32)
    out1d = _make_lookup(n_tokens, n_per)(tok2, weight)
    o5 = out1d.reshape(n_per, EMBEDDING_DIM // 8, n_tokens // IBLK, 8, IBLK)
    return o5.transpose(2, 4, 0, 1, 3).reshape(n_tokens, n_per, EMBEDDING_DIM)


# speedup vs baseline: 1.5540x; 1.4872x over previous
import functools

import jax
import jax.numpy as jnp
from jax import lax
from jax.experimental import pallas as pl
from jax.experimental.pallas import tpu as pltpu
from jax.experimental.pallas import tpu_sc as plsc

NUM_EMBEDDING = 1000000
EMBEDDING_DIM = 64
IBLK = 128
PITCH = IBLK + 1

_INFO = plsc.get_sparse_core_info()
_NC = _INFO.num_cores
_NS = _INFO.num_subcores
_NW = _NC * _NS


def _make_lookup(n_tokens, n_per):
    i_per_w = n_tokens // _NW
    nblk = i_per_w // IBLK
    n_iblk = n_tokens // IBLK
    n_jobs = n_per * nblk
    dtiles = EMBEDDING_DIM // 8
    out_rows = n_tokens * n_per * EMBEDDING_DIM // IBLK
    mesh = plsc.VectorSubcoreMesh(core_axis_name="c", subcore_axis_name="s")

    @functools.partial(
        pl.kernel,
        mesh=mesh,
        out_type=jax.ShapeDtypeStruct((out_rows, IBLK), jnp.float32),
        scratch_types=[
            pltpu.VMEM((n_per, i_per_w), jnp.int32),
            pltpu.VMEM((2, IBLK, EMBEDDING_DIM), jnp.float32),
            pltpu.VMEM((2, EMBEDDING_DIM, PITCH), jnp.float32),
            pltpu.SemaphoreType.DMA,
            pltpu.SemaphoreType.DMA,
        ],
        compiler_params=pltpu.CompilerParams(use_tc_tiling_on_sc=False,
                                             needs_layout_passes=False,
                                             disable_bounds_checks=True),
    )
    def lookup_kernel(tok_hbm, table_hbm, out_hbm, idx_v, rows_v, t_v,
                      gsem, wsem):
        wid = lax.axis_index("s") * _NC + lax.axis_index("c")
        i0w = wid * i_per_w
        pltpu.sync_copy(tok_hbm.at[:, pl.ds(i0w, i_per_w)], idx_v)
        lanes = lax.iota(jnp.int32, 16)

        def fire_gather(job, buf):
            j = lax.div(job, nblk)
            ib = lax.rem(job, nblk)
            pltpu.async_copy(
                table_hbm.at[idx_v.at[j, pl.ds(ib * IBLK, IBLK)]],
                rows_v.at[buf],
                gsem,
            )

        def drain_gather(buf):
            pltpu.make_async_copy(
                table_hbm.at[pl.ds(0, IBLK)],
                rows_v.at[buf],
                gsem,
            ).wait()

        def drain_writebacks():
            pltpu.make_async_copy(
                t_v.at[0, :, pl.ds(0, IBLK)],
                out_hbm.at[pl.ds(0, EMBEDDING_DIM)],
                wsem,
            ).wait()

        def transpose_into(buf):
            @plsc.parallel_loop(0, IBLK, 1, unroll=4)
            def _(t):
                tvec = jnp.full((16,), 0, jnp.int32) + t
                for dc in range(EMBEDDING_DIM // 16):
                    vals = rows_v[buf, t, pl.ds(dc * 16, 16)]
                    plsc.store_scatter(
                        t_v.at[buf],
                        [dc * 16 + lanes, tvec],
                        vals,
                    )

        def fire_writebacks(job, buf):
            j = lax.div(job, nblk)
            ib = lax.rem(job, nblk)
            gblk = wid * nblk + ib
            for db in range(dtiles):
                pltpu.async_copy(
                    t_v.at[buf, pl.ds(db * 8, 8), pl.ds(0, IBLK)],
                    out_hbm.at[pl.ds(((j * dtiles + db) * n_iblk + gblk) * 8,
                                     8)],
                    wsem,
                )

        fire_gather(0, 0)
        fire_writebacks(0, 1)

        def job_body(job, carry):
            cur = lax.rem(job, 2)
            nxt = 1 - cur
            drain_gather(cur)
            nxt_job = lax.min(job + 1, n_jobs - 1)
            fire_gather(nxt_job, nxt)
            transpose_into(cur)
            drain_writebacks()
            fire_writebacks(job, cur)
            return carry

        lax.fori_loop(0, n_jobs, job_body, 0)
        drain_gather(lax.rem(n_jobs, 2))
        drain_writebacks()

    return lookup_kernel


def kernel(token_ids, weight):
    n_tokens, n_per = token_ids.shape
    tok2 = token_ids.T.astype(jnp.int32)
    out2d = _make_lookup(n_tokens, n_per)(tok2, weight)
    o5 = out2d.reshape(n_per, EMBEDDING_DIM // 8, n_tokens // IBLK, 8, IBLK)
    return o5.transpose(2, 4, 0, 1, 3).reshape(n_tokens, n_per, EMBEDDING_DIM)


# speedup vs baseline: 1.7788x; 1.1447x over previous
import functools

import jax
import jax.numpy as jnp
from jax import lax
from jax.experimental import pallas as pl
from jax.experimental.pallas import tpu as pltpu
from jax.experimental.pallas import tpu_sc as plsc

NUM_EMBEDDING = 1000000
EMBEDDING_DIM = 64
IBLK = 128
PITCH = IBLK + 1

_INFO = plsc.get_sparse_core_info()
_NC = _INFO.num_cores
_NS = _INFO.num_subcores
_NW = _NC * _NS


def _make_lookup(n_tokens, n_per):
    i_per_w = n_tokens // _NW
    nblk = i_per_w // IBLK
    n_iblk = n_tokens // IBLK
    n_jobs = n_per * nblk
    dtiles = EMBEDDING_DIM // 8
    out_rows = n_tokens * n_per * EMBEDDING_DIM // IBLK
    mesh = plsc.VectorSubcoreMesh(core_axis_name="c", subcore_axis_name="s")

    @functools.partial(
        pl.kernel,
        mesh=mesh,
        out_type=jax.ShapeDtypeStruct((out_rows, IBLK), jnp.float32),
        scratch_types=[
            pltpu.VMEM((n_per, i_per_w), jnp.int32),
            pltpu.VMEM((2, IBLK, EMBEDDING_DIM), jnp.float32),
            pltpu.VMEM((2, EMBEDDING_DIM, PITCH), jnp.float32),
            pltpu.SemaphoreType.DMA,
            pltpu.SemaphoreType.DMA,
        ],
        compiler_params=pltpu.CompilerParams(use_tc_tiling_on_sc=False,
                                             needs_layout_passes=False,
                                             disable_bounds_checks=True),
    )
    def lookup_kernel(tok_hbm, table_hbm, out_hbm, idx_v, rows_v, t_v,
                      gsem, wsem):
        wid = lax.axis_index("s") * _NC + lax.axis_index("c")
        i0w = wid * i_per_w
        pltpu.sync_copy(tok_hbm.at[:, pl.ds(i0w, i_per_w)], idx_v)
        lanes = lax.iota(jnp.int32, 16)

        def fire_gather(job, buf):
            j = lax.div(job, nblk)
            ib = lax.rem(job, nblk)
            pltpu.async_copy(
                table_hbm.at[idx_v.at[j, pl.ds(ib * IBLK, IBLK)]],
                rows_v.at[buf],
                gsem,
            )

        def drain_gather(buf):
            pltpu.make_async_copy(
                table_hbm.at[pl.ds(0, IBLK)],
                rows_v.at[buf],
                gsem,
            ).wait()

        def drain_writebacks():
            pltpu.make_async_copy(
                t_v.at[0, :, pl.ds(0, IBLK)],
                out_hbm.at[pl.ds(0, EMBEDDING_DIM)],
                wsem,
            ).wait()

        def transpose_into(buf):
            @plsc.parallel_loop(0, IBLK, 1, unroll=4)
            def _(t):
                tvec = jnp.full((16,), 0, jnp.int32) + t
                for dc in range(EMBEDDING_DIM // 16):
                    vals = rows_v[buf, t, pl.ds(dc * 16, 16)]
                    plsc.store_scatter(
                        t_v.at[buf],
                        [dc * 16 + lanes, tvec],
                        vals,
                    )

        def fire_writebacks(job, buf):
            j = lax.div(job, nblk)
            ib = lax.rem(job, nblk)
            gblk = wid * nblk + ib
            for db in range(dtiles):
                pltpu.async_copy(
                    t_v.at[buf, pl.ds(db * 8, 8), pl.ds(0, IBLK)],
                    out_hbm.at[pl.ds(((j * dtiles + db) * n_iblk + gblk) * 8,
                                     8)],
                    wsem,
                )

        fire_gather(0, 0)
        fire_writebacks(0, 1)

        def job_body(job, carry):
            cur = lax.rem(job, 2)
            nxt = 1 - cur
            drain_gather(cur)
            nxt_job = lax.min(job + 1, n_jobs - 1)
            fire_gather(nxt_job, nxt)
            transpose_into(cur)
            drain_writebacks()
            fire_writebacks(job, cur)
            return carry

        lax.fori_loop(0, n_jobs, job_body, 0)
        drain_gather(lax.rem(n_jobs, 2))
        drain_writebacks()

    return lookup_kernel


def _make_table_format(n_rows):
    n_full = n_rows // IBLK
    tail = n_rows - n_full * IBLK
    quota = (n_full + _NW - 1) // _NW
    npair = (quota + 1) // 2
    mesh = plsc.VectorSubcoreMesh(core_axis_name="c", subcore_axis_name="s")

    @functools.partial(
        pl.kernel,
        mesh=mesh,
        out_type=jax.ShapeDtypeStruct((n_rows * EMBEDDING_DIM,), jnp.float32),
        scratch_types=[
            pltpu.VMEM((EMBEDDING_DIM, IBLK), jnp.float32),
            pltpu.VMEM((EMBEDDING_DIM, IBLK), jnp.float32),
            pltpu.VMEM((EMBEDDING_DIM * IBLK,), jnp.float32),
            pltpu.VMEM((EMBEDDING_DIM * IBLK,), jnp.float32),
            pltpu.VMEM((EMBEDDING_DIM, tail), jnp.float32),
            pltpu.VMEM((tail * EMBEDDING_DIM,), jnp.float32),
            pltpu.SemaphoreType.DMA,
            pltpu.SemaphoreType.DMA,
            pltpu.SemaphoreType.DMA,
            pltpu.SemaphoreType.DMA,
        ],
        compiler_params=pltpu.CompilerParams(use_tc_tiling_on_sc=True,
                                             needs_layout_passes=False,
                                             disable_bounds_checks=True),
    )
    def format_kernel(wt_hbm, out_hbm, src_a, src_b, tst_a, tst_b,
                      tl_v, tlst_v, gsem_a, gsem_b, wsem_a, wsem_b):
        wid = lax.axis_index("s") * _NC + lax.axis_index("c")
        lanes = lax.iota(jnp.int32, 16)
        l64 = lanes * 64
        srcs = (src_a, src_b)
        tsts = (tst_a, tst_b)
        gsems = (gsem_a, gsem_b)
        wsems = (wsem_a, wsem_b)

        def pos_of(t):
            return lax.min(wid * quota + t, n_full - 1)

        def out_slice(t):
            return out_hbm.at[pl.ds(pos_of(t) * IBLK * EMBEDDING_DIM,
                                    EMBEDDING_DIM * IBLK)]

        def fire_read(t, b):
            pltpu.async_copy(
                wt_hbm.at[:, pl.ds(pos_of(t) * IBLK, IBLK)], srcs[b],
                gsems[b],
            )

        def drain_read(b):
            pltpu.make_async_copy(
                wt_hbm.at[:, pl.ds(0, IBLK)], srcs[b], gsems[b]
            ).wait()

        def drain_writeback(b):
            pltpu.make_async_copy(
                tsts[b], out_hbm.at[pl.ds(0, EMBEDDING_DIM * IBLK)], wsems[b]
            ).wait()

        def transpose_pos(sv, tv, width):
            @plsc.parallel_loop(0, width, 1, unroll=2)
            def _(q):
                vb = (q // 16) * 16
                s = q % 16
                rel = (lanes + s) & 15
                vvec = vb + lanes
                dstbase = l64 + rel + vb * 64
                for dc in range(EMBEDDING_DIM // 16):
                    vals = plsc.load_gather(sv, [dc * 16 + rel, vvec])
                    plsc.store_scatter(tv, [dstbase + dc * 16], vals)

        def pos_body(t, carry):
            pltpu.sync_copy(wt_hbm.at[:, pl.ds(pos_of(t) * IBLK, IBLK)],
                            src_a)
            transpose_pos(src_a, tst_a, IBLK)
            pltpu.sync_copy(tst_a, out_slice(t))
            return carry

        lax.fori_loop(0, quota, pos_body, 0)

        pltpu.sync_copy(wt_hbm.at[:, pl.ds(n_full * IBLK, tail)], tl_v)
        transpose_pos(tl_v, tlst_v, tail)
        pltpu.sync_copy(
            tlst_v,
            out_hbm.at[pl.ds(n_full * IBLK * EMBEDDING_DIM,
                             tail * EMBEDDING_DIM)],
        )

    return format_kernel


def kernel(token_ids, weight):
    n_tokens, n_per = token_ids.shape
    n_rows = weight.shape[0]
    tok2 = token_ids.T.astype(jnp.int32)
    w1d = _make_table_format(n_rows)(weight.T)
    table2 = w1d.reshape(n_rows, EMBEDDING_DIM)
    out2d = _make_lookup(n_tokens, n_per)(tok2, table2)
    o5 = out2d.reshape(n_per, EMBEDDING_DIM // 8, n_tokens // IBLK, 8, IBLK)
    return o5.transpose(2, 4, 0, 1, 3).reshape(n_tokens, n_per, EMBEDDING_DIM)


# speedup vs baseline: 1.9847x; 1.1157x over previous
import functools

import jax
import jax.numpy as jnp
from jax import lax
from jax.experimental import pallas as pl
from jax.experimental.pallas import tpu as pltpu
from jax.experimental.pallas import tpu_sc as plsc

NUM_EMBEDDING = 1000000
EMBEDDING_DIM = 64
IBLK = 128
PITCH = IBLK + 1

_INFO = plsc.get_sparse_core_info()
_NC = _INFO.num_cores
_NS = _INFO.num_subcores
_NW = _NC * _NS


def _make_lookup(n_tokens, n_per):
    i_per_w = n_tokens // _NW
    nblk = i_per_w // IBLK
    n_iblk = n_tokens // IBLK
    n_jobs = n_per * nblk
    dtiles = EMBEDDING_DIM // 8
    out_rows = n_tokens * n_per * EMBEDDING_DIM // IBLK
    mesh = plsc.VectorSubcoreMesh(core_axis_name="c", subcore_axis_name="s")

    @functools.partial(
        pl.kernel,
        mesh=mesh,
        out_type=jax.ShapeDtypeStruct((out_rows, IBLK), jnp.float32),
        scratch_types=[
            pltpu.VMEM((n_per, i_per_w), jnp.int32),
            pltpu.VMEM((2, IBLK, EMBEDDING_DIM), jnp.float32),
            pltpu.VMEM((2, EMBEDDING_DIM, PITCH), jnp.float32),
            pltpu.SemaphoreType.DMA,
            pltpu.SemaphoreType.DMA,
        ],
        compiler_params=pltpu.CompilerParams(use_tc_tiling_on_sc=False,
                                             needs_layout_passes=False,
                                             disable_bounds_checks=True),
    )
    def lookup_kernel(tok_hbm, table_hbm, out_hbm, idx_v, rows_v, t_v,
                      gsem, wsem):
        wid = lax.axis_index("s") * _NC + lax.axis_index("c")
        i0w = wid * i_per_w
        pltpu.sync_copy(tok_hbm.at[:, pl.ds(i0w, i_per_w)], idx_v)
        lanes = lax.iota(jnp.int32, 16)

        def fire_gather(job, buf):
            j = lax.div(job, nblk)
            ib = lax.rem(job, nblk)
            pltpu.async_copy(
                table_hbm.at[idx_v.at[j, pl.ds(ib * IBLK, IBLK)]],
                rows_v.at[buf],
                gsem,
            )

        def drain_gather(buf):
            pltpu.make_async_copy(
                table_hbm.at[pl.ds(0, IBLK)],
                rows_v.at[buf],
                gsem,
            ).wait()

        def drain_writebacks():
            pltpu.make_async_copy(
                t_v.at[0, :, pl.ds(0, IBLK)],
                out_hbm.at[pl.ds(0, EMBEDDING_DIM)],
                wsem,
            ).wait()

        def transpose_into(buf):
            @plsc.parallel_loop(0, IBLK, 1, unroll=4)
            def _(t):
                tvec = jnp.full((16,), 0, jnp.int32) + t
                for dc in range(EMBEDDING_DIM // 16):
                    vals = rows_v[buf, t, pl.ds(dc * 16, 16)]
                    plsc.store_scatter(
                        t_v.at[buf],
                        [dc * 16 + lanes, tvec],
                        vals,
                    )

        def fire_writebacks(job, buf):
            j = lax.div(job, nblk)
            ib = lax.rem(job, nblk)
            gblk = wid * nblk + ib
            for db in range(dtiles):
                pltpu.async_copy(
                    t_v.at[buf, pl.ds(db * 8, 8), pl.ds(0, IBLK)],
                    out_hbm.at[pl.ds(((j * dtiles + db) * n_iblk + gblk) * 8,
                                     8)],
                    wsem,
                )

        fire_gather(0, 0)
        fire_writebacks(0, 1)

        def job_body(job, carry):
            cur = lax.rem(job, 2)
            nxt = 1 - cur
            drain_gather(cur)
            nxt_job = lax.min(job + 1, n_jobs - 1)
            fire_gather(nxt_job, nxt)
            transpose_into(cur)
            drain_writebacks()
            fire_writebacks(job, cur)
            return carry

        lax.fori_loop(0, n_jobs, job_body, 0)
        drain_gather(lax.rem(n_jobs, 2))
        drain_writebacks()

    return lookup_kernel


def _make_table_format(n_rows):
    n_full = n_rows // IBLK
    tail = n_rows - n_full * IBLK
    quota = (n_full + _NW - 1) // _NW
    npair = (quota + 1) // 2
    mesh = plsc.VectorSubcoreMesh(core_axis_name="c", subcore_axis_name="s")

    @functools.partial(
        pl.kernel,
        mesh=mesh,
        out_type=jax.ShapeDtypeStruct((n_rows * EMBEDDING_DIM,), jnp.float32),
        scratch_types=[
            pltpu.VMEM((EMBEDDING_DIM, IBLK), jnp.float32),
            pltpu.VMEM((EMBEDDING_DIM, IBLK), jnp.float32),
            pltpu.VMEM((EMBEDDING_DIM * IBLK,), jnp.float32),
            pltpu.VMEM((EMBEDDING_DIM * IBLK,), jnp.float32),
            pltpu.VMEM((EMBEDDING_DIM, tail), jnp.float32),
            pltpu.VMEM((tail * EMBEDDING_DIM,), jnp.float32),
            pltpu.SemaphoreType.DMA,
            pltpu.SemaphoreType.DMA,
            pltpu.SemaphoreType.DMA,
            pltpu.SemaphoreType.DMA,
        ],
        compiler_params=pltpu.CompilerParams(use_tc_tiling_on_sc=True,
                                             needs_layout_passes=False,
                                             disable_bounds_checks=True),
    )
    def format_kernel(wt_hbm, out_hbm, src_a, src_b, tst_a, tst_b,
                      tl_v, tlst_v, gsem_a, gsem_b, wsem_a, wsem_b):
        wid = lax.axis_index("s") * _NC + lax.axis_index("c")
        lanes = lax.iota(jnp.int32, 16)
        l64 = lanes * 64
        srcs = (src_a, src_b)
        tsts = (tst_a, tst_b)
        gsems = (gsem_a, gsem_b)
        wsems = (wsem_a, wsem_b)

        def pos_of(t):
            return lax.min(wid * quota + t, n_full - 1)

        def out_slice(t):
            return out_hbm.at[pl.ds(pos_of(t) * IBLK * EMBEDDING_DIM,
                                    EMBEDDING_DIM * IBLK)]

        def fire_read(t, b):
            pltpu.async_copy(
                wt_hbm.at[:, pl.ds(pos_of(t) * IBLK, IBLK)], srcs[b],
                gsems[b],
            )

        def drain_read(b):
            pltpu.make_async_copy(
                wt_hbm.at[:, pl.ds(0, IBLK)], srcs[b], gsems[b]
            ).wait()

        def drain_writeback(b):
            pltpu.make_async_copy(
                tsts[b], out_hbm.at[pl.ds(0, EMBEDDING_DIM * IBLK)], wsems[b]
            ).wait()

        def transpose_pos(sv, tv, width):
            @plsc.parallel_loop(0, width, 1, unroll=2)
            def _(q):
                vb = (q // 16) * 16
                s = q % 16
                rel = (lanes + s) & 15
                vvec = vb + lanes
                dstbase = l64 + rel + vb * 64
                for dc in range(EMBEDDING_DIM // 16):
                    vals = plsc.load_gather(sv, [dc * 16 + rel, vvec])
                    plsc.store_scatter(tv, [dstbase + dc * 16], vals)

        pltpu.async_copy(tst_a, out_slice(0), wsem_a)
        pltpu.async_copy(tst_b, out_slice(1), wsem_b)

        def pair_body(i, carry):
            for b in range(2):
                t = i * 2 + b
                pltpu.sync_copy(wt_hbm.at[:, pl.ds(pos_of(t) * IBLK, IBLK)],
                                srcs[b])
                drain_writeback(b)
                transpose_pos(srcs[b], tsts[b], IBLK)
                pltpu.async_copy(tsts[b], out_slice(t), wsems[b])
            return carry

        lax.fori_loop(0, npair, pair_body, 0)
        drain_writeback(0)
        drain_writeback(1)

        pltpu.sync_copy(wt_hbm.at[:, pl.ds(n_full * IBLK, tail)], tl_v)
        transpose_pos(tl_v, tlst_v, tail)
        pltpu.sync_copy(
            tlst_v,
            out_hbm.at[pl.ds(n_full * IBLK * EMBEDDING_DIM,
                             tail * EMBEDDING_DIM)],
        )

    return format_kernel


def kernel(token_ids, weight):
    n_tokens, n_per = token_ids.shape
    n_rows = weight.shape[0]
    tok2 = token_ids.T.astype(jnp.int32)
    w1d = _make_table_format(n_rows)(weight.T)
    table2 = w1d.reshape(n_rows, EMBEDDING_DIM)
    out2d = _make_lookup(n_tokens, n_per)(tok2, table2)
    o5 = out2d.reshape(n_per, EMBEDDING_DIM // 8, n_tokens // IBLK, 8, IBLK)
    return o5.transpose(2, 4, 0, 1, 3).reshape(n_tokens, n_per, EMBEDDING_DIM)
